# Initial kernel scaffold; baseline (speedup 1.0000x reference)
import jax
import jax.numpy as jnp
from jax.experimental import pallas as pl


def kernel(user_index, item_index, adj_u2i, adj_i2u, adj_entity, adj_relation, user_emb, entity_emb, W_R, W_user_agg, b_user_agg, W_kg_agg, b_kg_agg, c):
    raise NotImplementedError("write your pallas kernel here")


# speedup vs baseline: 1.7711x; 1.7711x over previous
import functools

import jax
import jax.numpy as jnp
from jax import lax
from jax.experimental import pallas as pl
from jax.experimental.pallas import tpu as pltpu
from jax.experimental.pallas import tpu_sc as plsc

DIM = 32
S = 4


def _expand4(src, dst, n):
  for c in range(n // 16):
    v4 = src[pl.ds(c * 16, 16)] * 4
    for k in range(S):
      dst[pl.ds(k * n + c * 16, 16)] = v4 + k


def _make_gather(B, nw):
  bp = B // nw

  mesh = plsc.VectorSubcoreMesh(core_axis_name="c", subcore_axis_name="s")

  @functools.partial(
      pl.kernel,
      mesh=mesh,
      compiler_params=pltpu.CompilerParams(use_tc_tiling_on_sc=False),
      out_type=[
          jax.ShapeDtypeStruct((B, DIM), jnp.float32),
          jax.ShapeDtypeStruct((S, B, DIM), jnp.float32),
          jax.ShapeDtypeStruct((S, B, DIM), jnp.float32),
          jax.ShapeDtypeStruct((B, DIM), jnp.float32),
          jax.ShapeDtypeStruct((S, B, DIM), jnp.float32),
          jax.ShapeDtypeStruct((S, S, B, DIM), jnp.float32),
          jax.ShapeDtypeStruct((S, B), jnp.int32),
          jax.ShapeDtypeStruct((S, S, B), jnp.int32),
      ],
      scratch_types=[
          pltpu.VMEM((bp,), jnp.int32),
          pltpu.VMEM((bp,), jnp.int32),
          pltpu.VMEM((bp * S,), jnp.int32),
          pltpu.VMEM((bp * S,), jnp.int32),
          pltpu.VMEM((bp * S,), jnp.int32),
          pltpu.VMEM((bp * S,), jnp.int32),
          pltpu.VMEM((bp * S * S,), jnp.int32),
          pltpu.VMEM((bp * S * S,), jnp.int32),
          pltpu.VMEM((bp * S * S,), jnp.int32),
          pltpu.VMEM((bp, DIM), jnp.float32),
          pltpu.VMEM((bp * S, DIM), jnp.float32),
          pltpu.VMEM((bp * S * S, DIM), jnp.float32),
          pltpu.SemaphoreType.DMA,
          pltpu.SemaphoreType.DMA,
      ],
  )
  def gather(ui_hbm, ii_hbm, u2i_hbm, i2u_hbm, ae_hbm, ar_hbm, ue_hbm, ee_hbm,
             a_out, en_out, u_out, e0_out, e1_out, e2_out, r0_out, r1_out,
             ui_v, ii_v, exp128, val128, idx1_v, r0_v, exp512, idx2_v, r1_v,
             rows32, rows128, rows512, sem, osem):
    wid = lax.axis_index("s") * 2 + lax.axis_index("c")
    base = wid * bp

    pltpu.sync_copy(ui_hbm.at[pl.ds(base, bp)], ui_v)
    pltpu.sync_copy(ii_hbm.at[pl.ds(base, bp)], ii_v)

    pltpu.async_copy(ue_hbm.at[ui_v], rows32, sem).wait()
    pltpu.sync_copy(rows32, a_out.at[pl.ds(base, bp)])

    _expand4(ui_v, exp128, bp)
    pltpu.async_copy(u2i_hbm.at[exp128], val128, sem).wait()
    pltpu.async_copy(ee_hbm.at[val128], rows128, sem).wait()
    cps = [pltpu.async_copy(rows128.at[pl.ds(k * bp, bp)],
                            en_out.at[k, pl.ds(base, bp)], osem)
           for k in range(S)]
    for cp in cps:
      cp.wait()

    _expand4(ii_v, exp128, bp)
    pltpu.async_copy(i2u_hbm.at[exp128], val128, sem).wait()
    pltpu.async_copy(ue_hbm.at[val128], rows128, sem).wait()
    cps = [pltpu.async_copy(rows128.at[pl.ds(k * bp, bp)],
                            u_out.at[k, pl.ds(base, bp)], osem)
           for k in range(S)]
    for cp in cps:
      cp.wait()

    pltpu.async_copy(ee_hbm.at[ii_v], rows32, sem).wait()
    pltpu.sync_copy(rows32, e0_out.at[pl.ds(base, bp)])

    pltpu.async_copy(ae_hbm.at[exp128], idx1_v, sem).wait()
    pltpu.async_copy(ar_hbm.at[exp128], r0_v, sem).wait()
    cps = [pltpu.async_copy(r0_v.at[pl.ds(k * bp, bp)],
                            r0_out.at[k, pl.ds(base, bp)], osem)
           for k in range(S)]
    pltpu.async_copy(ee_hbm.at[idx1_v], rows128, sem).wait()
    cps += [pltpu.async_copy(rows128.at[pl.ds(s * bp, bp)],
                             e1_out.at[s, pl.ds(base, bp)], osem)
            for s in range(S)]
    for cp in cps:
      cp.wait()

    _expand4(idx1_v, exp512, bp * S)
    pltpu.async_copy(ae_hbm.at[exp512], idx2_v, sem).wait()
    pltpu.async_copy(ar_hbm.at[exp512], r1_v, sem).wait()
    cps = [pltpu.async_copy(r1_v.at[pl.ds((k * S + s) * bp, bp)],
                            r1_out.at[k, s, pl.ds(base, bp)], osem)
           for k in range(S) for s in range(S)]
    pltpu.async_copy(ee_hbm.at[idx2_v], rows512, sem).wait()
    cps += [pltpu.async_copy(rows512.at[pl.ds((k * S + s) * bp, bp)],
                             e2_out.at[k, s, pl.ds(base, bp)], osem)
            for k in range(S) for s in range(S)]
    for cp in cps:
      cp.wait()

  return gather


def _soft4(logits):
  mx = jnp.maximum(jnp.maximum(logits[0], logits[1]),
                   jnp.maximum(logits[2], logits[3]))
  es = [jnp.exp(l - mx) for l in logits]
  tot = es[0] + es[1] + es[2] + es[3]
  return [e / tot for e in es]


def _dotk(x, y):
  return jnp.sum(x * y, axis=-1, keepdims=True)


def _wsum(att, vs):
  return att[0] * vs[0] + att[1] * vs[1] + att[2] * vs[2] + att[3] * vs[3]


def _tc_body(a_ref, en_ref, u_ref, e0_ref, e1_ref, e2_ref, rf_ref, wr_ref,
             wu_ref, wk_ref, bu_ref, bk_ref, out_ref):
  f32 = jnp.float32
  Bb = a_ref.shape[0]
  col = lax.broadcasted_iota(jnp.int32, (1, DIM), 1)
  m = (col != 0).astype(f32)
  W16 = wr_ref[16 * DIM:17 * DIM, :]
  wu = wu_ref[...]
  wk = wk_ref[...]
  bu = bu_ref[...]
  bk = bk_ref[...]

  u_t = a_ref[...] * m
  n_ts = [jnp.dot(en_ref[s] * m, W16, preferred_element_type=f32)
          for s in range(S)]
  att = _soft4([_dotk(u_t, n) for n in n_ts])
  ngh = _wsum(att, n_ts)
  ue = jnp.tanh(jnp.dot(u_t + ngh, wu, preferred_element_type=f32) + bu) * m

  i_t = e0_ref[...] * m
  ungh = [jnp.dot(u_ref[s] * m, W16, preferred_element_type=f32)
          for s in range(S)]

  e1s = [e1_ref[s] * m for s in range(S)]
  e2s = [e2_ref[k, s] * m for k in range(S) for s in range(S)]
  X = jnp.concatenate(e1s + e2s, axis=0)
  rvec = jnp.concatenate(
      [rf_ref[t].reshape(Bb, 1) for t in range(20)], axis=0)
  acc = jnp.zeros_like(X)
  for r in range(16):
    pr = jnp.dot(X, wr_ref[DIM * r:DIM * (r + 1), :], preferred_element_type=f32)
    acc = acc + (rvec == float(r)).astype(f32) * pr
  etw0 = [acc[s * Bb:(s + 1) * Bb] for s in range(S)]
  etw1 = [[acc[(S + S * k + s) * Bb:(S + S * k + s + 1) * Bb] for k in range(S)]
          for s in range(S)]

  a0 = _soft4([_dotk(i_t, e) for e in etw0])
  comb0 = i_t + _wsum(a0, etw0)
  comb1 = []
  for s in range(S):
    an = _soft4([_dotk(i_t, e) for e in etw1[s]])
    comb1.append(e1s[s] + _wsum(an, etw1[s]))
  C = jnp.concatenate([comb0] + comb1, axis=0)
  H = jax.nn.relu(jnp.dot(C, wk, preferred_element_type=f32) + bk)
  v0 = H[:Bb]
  v1 = [H[(1 + s) * Bb:(2 + s) * Bb] for s in range(S)]

  X2 = jnp.concatenate([v * m for v in v1], axis=0)
  rvec2 = jnp.concatenate(
      [rf_ref[t].reshape(Bb, 1) for t in range(S)], axis=0)
  acc2 = jnp.zeros_like(X2)
  for r in range(16):
    pr = jnp.dot(X2, wr_ref[DIM * r:DIM * (r + 1), :], preferred_element_type=f32)
    acc2 = acc2 + (rvec2 == float(r)).astype(f32) * pr
  etw2 = [acc2[s * Bb:(s + 1) * Bb] for s in range(S)]

  au = _soft4([_dotk(i_t, un) for un in ungh])
  user_agg = _wsum(au, ungh)

  a2 = _soft4([_dotk(i_t, e) for e in etw2])
  comb = v0 * m + _wsum(a2, etw2) + user_agg
  ie = jnp.tanh(jnp.dot(comb, wk, preferred_element_type=f32) + bk) * m

  score = jax.nn.sigmoid(_dotk(ue, ie))
  score = jnp.clip(score, 1e-6, 1e6)
  score = jnp.where(jnp.isnan(score), 0.0, score)
  out_ref[...] = score


def _make_compute(B, Bb):
  nb = B // Bb
  full = lambda shape: pl.BlockSpec(shape, lambda i: tuple(0 for _ in shape))
  return pl.pallas_call(
      _tc_body,
      grid=(nb,),
      in_specs=[
          pl.BlockSpec((Bb, DIM), lambda i: (i, 0)),
          pl.BlockSpec((S, Bb, DIM), lambda i: (0, i, 0)),
          pl.BlockSpec((S, Bb, DIM), lambda i: (0, i, 0)),
          pl.BlockSpec((Bb, DIM), lambda i: (i, 0)),
          pl.BlockSpec((S, Bb, DIM), lambda i: (0, i, 0)),
          pl.BlockSpec((S, S, Bb, DIM), lambda i: (0, 0, i, 0)),
          pl.BlockSpec((24, Bb), lambda i: (0, i)),
          full((17 * DIM, DIM)),
          full((DIM, DIM)),
          full((DIM, DIM)),
          full((1, DIM)),
          full((1, DIM)),
      ],
      out_specs=pl.BlockSpec((Bb, 1), lambda i: (i, 0)),
      out_shape=jax.ShapeDtypeStruct((B, 1), jnp.float32),
  )


def kernel(user_index, item_index, adj_u2i, adj_i2u, adj_entity, adj_relation,
           user_emb, entity_emb, W_R, W_user_agg, b_user_agg, W_kg_agg,
           b_kg_agg, c):
  B = user_index.shape[0]

  ui = user_index.astype(jnp.int32)
  ii = item_index.astype(jnp.int32)
  u2i_f = adj_u2i.astype(jnp.int32).reshape(-1)
  i2u_f = adj_i2u.astype(jnp.int32).reshape(-1)
  ae_f = adj_entity.astype(jnp.int32).reshape(-1)
  ar_f = adj_relation.astype(jnp.int32).reshape(-1)

  info = plsc.get_sparse_core_info()
  nw = info.num_cores * info.num_subcores

  gather = _make_gather(B, nw)
  A, EN, U, E0, E1, E2, R0, R1 = gather(
      ui, ii, u2i_f, i2u_f, ae_f, ar_f,
      user_emb.astype(jnp.float32), entity_emb.astype(jnp.float32))

  Rf = jnp.concatenate([R0, R1.reshape(S * S, B)], axis=0).astype(jnp.float32)
  Rf = jnp.pad(Rf, ((0, 4), (0, 0)))

  compute = _make_compute(B, 128)
  score = compute(A, EN, U, E0, E1, E2, Rf,
                  W_R.astype(jnp.float32).reshape(17 * DIM, DIM),
                  W_user_agg.astype(jnp.float32),
                  W_kg_agg.astype(jnp.float32),
                  b_user_agg.astype(jnp.float32).reshape(1, DIM),
                  b_kg_agg.astype(jnp.float32).reshape(1, DIM))
  return score.reshape(B)


# speedup vs baseline: 3.6267x; 2.0476x over previous
import functools

import jax
import jax.numpy as jnp
from jax import lax
from jax.experimental import pallas as pl
from jax.experimental.pallas import tpu as pltpu
from jax.experimental.pallas import tpu_sc as plsc

DIM = 32
S = 4


def _expand4(src, dst, n, N):
  for c in range(n // 16):
    v = src[pl.ds(c * 16, 16)]
    for k in range(S):
      dst[pl.ds(k * n + c * 16, 16)] = v + (k * N)


def _make_gather(B, nw, n_user, n_item, n_ent):
  bp = B // nw

  mesh = plsc.VectorSubcoreMesh(core_axis_name="c", subcore_axis_name="s")

  @functools.partial(
      pl.kernel,
      mesh=mesh,
      compiler_params=pltpu.CompilerParams(use_tc_tiling_on_sc=False),
      out_type=[
          jax.ShapeDtypeStruct((B, DIM), jnp.float32),
          jax.ShapeDtypeStruct((S, B, DIM), jnp.float32),
          jax.ShapeDtypeStruct((S, B, DIM), jnp.float32),
          jax.ShapeDtypeStruct((B, DIM), jnp.float32),
          jax.ShapeDtypeStruct((S, B, DIM), jnp.float32),
          jax.ShapeDtypeStruct((S, S, B, DIM), jnp.float32),
          jax.ShapeDtypeStruct((S, B), jnp.int32),
          jax.ShapeDtypeStruct((S, S, B), jnp.int32),
      ],
      scratch_types=[
          pltpu.VMEM((bp,), jnp.int32),
          pltpu.VMEM((bp,), jnp.int32),
          pltpu.VMEM((bp * S,), jnp.int32),
          pltpu.VMEM((bp * S,), jnp.int32),
          pltpu.VMEM((bp * S,), jnp.int32),
          pltpu.VMEM((bp * S,), jnp.int32),
          pltpu.VMEM((bp * S * S,), jnp.int32),
          pltpu.VMEM((bp * S * S,), jnp.int32),
          pltpu.VMEM((bp * S * S,), jnp.int32),
          pltpu.VMEM((bp, DIM), jnp.float32),
          pltpu.VMEM((bp * S, DIM), jnp.float32),
          pltpu.VMEM((bp * S * S, DIM), jnp.float32),
          pltpu.SemaphoreType.DMA,
          pltpu.SemaphoreType.DMA,
      ],
  )
  def gather(ui_hbm, ii_hbm, u2i_hbm, i2u_hbm, ae_hbm, ar_hbm, ue_hbm, ee_hbm,
             a_out, en_out, u_out, e0_out, e1_out, e2_out, r0_out, r1_out,
             ui_v, ii_v, exp128, val128, idx1_v, r0_v, exp512, idx2_v, r1_v,
             rows32, rows128, rows512, sem, osem):
    wid = lax.axis_index("s") * 2 + lax.axis_index("c")
    base = wid * bp

    pltpu.sync_copy(ui_hbm.at[pl.ds(base, bp)], ui_v)
    pltpu.sync_copy(ii_hbm.at[pl.ds(base, bp)], ii_v)

    pltpu.async_copy(ue_hbm.at[ui_v], rows32, sem).wait()
    pltpu.sync_copy(rows32, a_out.at[pl.ds(base, bp)])

    _expand4(ui_v, exp128, bp, n_user)
    pltpu.async_copy(u2i_hbm.at[exp128], val128, sem).wait()
    pltpu.async_copy(ee_hbm.at[val128], rows128, sem).wait()
    cps = [pltpu.async_copy(rows128.at[pl.ds(k * bp, bp)],
                            en_out.at[k, pl.ds(base, bp)], osem)
           for k in range(S)]
    for cp in cps:
      cp.wait()

    _expand4(ii_v, exp128, bp, n_item)
    pltpu.async_copy(i2u_hbm.at[exp128], val128, sem).wait()
    pltpu.async_copy(ue_hbm.at[val128], rows128, sem).wait()
    cps = [pltpu.async_copy(rows128.at[pl.ds(k * bp, bp)],
                            u_out.at[k, pl.ds(base, bp)], osem)
           for k in range(S)]
    for cp in cps:
      cp.wait()

    pltpu.async_copy(ee_hbm.at[ii_v], rows32, sem).wait()
    pltpu.sync_copy(rows32, e0_out.at[pl.ds(base, bp)])

    _expand4(ii_v, exp128, bp, n_ent)
    pltpu.async_copy(ae_hbm.at[exp128], idx1_v, sem).wait()
    pltpu.async_copy(ar_hbm.at[exp128], r0_v, sem).wait()
    cps = [pltpu.async_copy(r0_v.at[pl.ds(k * bp, bp)],
                            r0_out.at[k, pl.ds(base, bp)], osem)
           for k in range(S)]
    pltpu.async_copy(ee_hbm.at[idx1_v], rows128, sem).wait()
    cps += [pltpu.async_copy(rows128.at[pl.ds(s * bp, bp)],
                             e1_out.at[s, pl.ds(base, bp)], osem)
            for s in range(S)]
    for cp in cps:
      cp.wait()

    _expand4(idx1_v, exp512, bp * S, n_ent)
    pltpu.async_copy(ae_hbm.at[exp512], idx2_v, sem).wait()
    pltpu.async_copy(ar_hbm.at[exp512], r1_v, sem).wait()
    cps = [pltpu.async_copy(r1_v.at[pl.ds((k * S + s) * bp, bp)],
                            r1_out.at[k, s, pl.ds(base, bp)], osem)
           for k in range(S) for s in range(S)]
    pltpu.async_copy(ee_hbm.at[idx2_v], rows512, sem).wait()
    cps += [pltpu.async_copy(rows512.at[pl.ds((k * S + s) * bp, bp)],
                             e2_out.at[k, s, pl.ds(base, bp)], osem)
            for k in range(S) for s in range(S)]
    for cp in cps:
      cp.wait()

  return gather


def _soft4(logits):
  mx = jnp.maximum(jnp.maximum(logits[0], logits[1]),
                   jnp.maximum(logits[2], logits[3]))
  es = [jnp.exp(l - mx) for l in logits]
  tot = es[0] + es[1] + es[2] + es[3]
  return [e / tot for e in es]


def _dotk(x, y):
  return jnp.sum(x * y, axis=-1, keepdims=True)


def _wsum(att, vs):
  return att[0] * vs[0] + att[1] * vs[1] + att[2] * vs[2] + att[3] * vs[3]


def _tc_body(a_ref, en_ref, u_ref, e0_ref, e1_ref, e2_ref, rf_ref, wr_ref,
             wu_ref, wk_ref, bu_ref, bk_ref, out_ref):
  f32 = jnp.float32
  Bb = a_ref.shape[0]
  col = lax.broadcasted_iota(jnp.int32, (1, DIM), 1)
  m = (col != 0).astype(f32)
  W16 = wr_ref[16 * DIM:17 * DIM, :]
  wu = wu_ref[...]
  wk = wk_ref[...]
  bu = bu_ref[...]
  bk = bk_ref[...]

  u_t = a_ref[...] * m
  n_ts = [jnp.dot(en_ref[s] * m, W16, preferred_element_type=f32)
          for s in range(S)]
  att = _soft4([_dotk(u_t, n) for n in n_ts])
  ngh = _wsum(att, n_ts)
  ue = jnp.tanh(jnp.dot(u_t + ngh, wu, preferred_element_type=f32) + bu) * m

  i_t = e0_ref[...] * m
  ungh = [jnp.dot(u_ref[s] * m, W16, preferred_element_type=f32)
          for s in range(S)]

  e1s = [e1_ref[s] * m for s in range(S)]
  e2s = [e2_ref[k, s] * m for k in range(S) for s in range(S)]
  X = jnp.concatenate(e1s + e2s, axis=0)
  rvec = jnp.concatenate(
      [rf_ref[t].reshape(Bb, 1) for t in range(20)], axis=0)
  acc = jnp.zeros_like(X)
  for r in range(16):
    pr = jnp.dot(X, wr_ref[DIM * r:DIM * (r + 1), :], preferred_element_type=f32)
    acc = acc + (rvec == float(r)).astype(f32) * pr
  etw0 = [acc[s * Bb:(s + 1) * Bb] for s in range(S)]
  etw1 = [[acc[(S + S * k + s) * Bb:(S + S * k + s + 1) * Bb] for k in range(S)]
          for s in range(S)]

  a0 = _soft4([_dotk(i_t, e) for e in etw0])
  comb0 = i_t + _wsum(a0, etw0)
  comb1 = []
  for s in range(S):
    an = _soft4([_dotk(i_t, e) for e in etw1[s]])
    comb1.append(e1s[s] + _wsum(an, etw1[s]))
  C = jnp.concatenate([comb0] + comb1, axis=0)
  H = jax.nn.relu(jnp.dot(C, wk, preferred_element_type=f32) + bk)
  v0 = H[:Bb]
  v1 = [H[(1 + s) * Bb:(2 + s) * Bb] for s in range(S)]

  X2 = jnp.concatenate([v * m for v in v1], axis=0)
  rvec2 = jnp.concatenate(
      [rf_ref[t].reshape(Bb, 1) for t in range(S)], axis=0)
  acc2 = jnp.zeros_like(X2)
  for r in range(16):
    pr = jnp.dot(X2, wr_ref[DIM * r:DIM * (r + 1), :], preferred_element_type=f32)
    acc2 = acc2 + (rvec2 == float(r)).astype(f32) * pr
  etw2 = [acc2[s * Bb:(s + 1) * Bb] for s in range(S)]

  au = _soft4([_dotk(i_t, un) for un in ungh])
  user_agg = _wsum(au, ungh)

  a2 = _soft4([_dotk(i_t, e) for e in etw2])
  comb = v0 * m + _wsum(a2, etw2) + user_agg
  ie = jnp.tanh(jnp.dot(comb, wk, preferred_element_type=f32) + bk) * m

  score = jax.nn.sigmoid(_dotk(ue, ie))
  score = jnp.clip(score, 1e-6, 1e6)
  score = jnp.where(jnp.isnan(score), 0.0, score)
  out_ref[...] = score


def _make_compute(B, Bb):
  nb = B // Bb
  full = lambda shape: pl.BlockSpec(shape, lambda i: tuple(0 for _ in shape))
  return pl.pallas_call(
      _tc_body,
      grid=(nb,),
      in_specs=[
          pl.BlockSpec((Bb, DIM), lambda i: (i, 0)),
          pl.BlockSpec((S, Bb, DIM), lambda i: (0, i, 0)),
          pl.BlockSpec((S, Bb, DIM), lambda i: (0, i, 0)),
          pl.BlockSpec((Bb, DIM), lambda i: (i, 0)),
          pl.BlockSpec((S, Bb, DIM), lambda i: (0, i, 0)),
          pl.BlockSpec((S, S, Bb, DIM), lambda i: (0, 0, i, 0)),
          pl.BlockSpec((24, Bb), lambda i: (0, i)),
          full((17 * DIM, DIM)),
          full((DIM, DIM)),
          full((DIM, DIM)),
          full((1, DIM)),
          full((1, DIM)),
      ],
      out_specs=pl.BlockSpec((Bb, 1), lambda i: (i, 0)),
      out_shape=jax.ShapeDtypeStruct((B, 1), jnp.float32),
  )


def kernel(user_index, item_index, adj_u2i, adj_i2u, adj_entity, adj_relation,
           user_emb, entity_emb, W_R, W_user_agg, b_user_agg, W_kg_agg,
           b_kg_agg, c):
  B = user_index.shape[0]

  ui = user_index.astype(jnp.int32)
  ii = item_index.astype(jnp.int32)
  u2i_f = adj_u2i.astype(jnp.int32).T.reshape(-1)
  i2u_f = adj_i2u.astype(jnp.int32).T.reshape(-1)
  ae_f = adj_entity.astype(jnp.int32).T.reshape(-1)
  ar_f = adj_relation.astype(jnp.int32).T.reshape(-1)

  info = plsc.get_sparse_core_info()
  nw = info.num_cores * info.num_subcores

  gather = _make_gather(B, nw, adj_u2i.shape[0], adj_i2u.shape[0],
                        adj_entity.shape[0])
  A, EN, U, E0, E1, E2, R0, R1 = gather(
      ui, ii, u2i_f, i2u_f, ae_f, ar_f,
      user_emb.astype(jnp.float32), entity_emb.astype(jnp.float32))

  Rf = jnp.concatenate([R0, R1.reshape(S * S, B)], axis=0).astype(jnp.float32)
  Rf = jnp.pad(Rf, ((0, 4), (0, 0)))

  compute = _make_compute(B, 128)
  score = compute(A, EN, U, E0, E1, E2, Rf,
                  W_R.astype(jnp.float32).reshape(17 * DIM, DIM),
                  W_user_agg.astype(jnp.float32),
                  W_kg_agg.astype(jnp.float32),
                  b_user_agg.astype(jnp.float32).reshape(1, DIM),
                  b_kg_agg.astype(jnp.float32).reshape(1, DIM))
  return score.reshape(B)
